# Initial kernel scaffold; baseline (speedup 1.0000x reference)
import jax
import jax.numpy as jnp
from jax.experimental import pallas as pl


def kernel(x, W_in, b_in, W_mid0, b_mid0, W_mid1, b_mid1, W_out, b_out, indx_seqs):
    raise NotImplementedError("write your pallas kernel here")


# speedup vs baseline: 1.1902x; 1.1902x over previous
import functools

import jax
import jax.numpy as jnp
from jax import lax
from jax.experimental import pallas as pl
from jax.experimental.pallas import tpu as pltpu
from jax.experimental.pallas import tpu_sc as plsc

NUM_IN = 1024
NUM_OUT = 1024
NUM_MID = 4096
FAN_IN = 16
BATCH = 1024

NC = 2
NS = 16
NW = NC * NS
RPW = NUM_MID // NW
L = 16
NCHUNK = BATCH // L


def _mm1_body(w_ref, x_ref, b_ref, o_ref):
    acc = lax.dot_general(w_ref[...], x_ref[...], (((1,), (1,)), ((), ())),
                          preferred_element_type=jnp.float32)
    o_ref[...] = jnp.maximum(acc + b_ref[...], 0.0)


def _mm1(W_in, x, b_in):
    BM = 1024
    return pl.pallas_call(
        _mm1_body,
        grid=(NUM_MID // BM,),
        in_specs=[
            pl.BlockSpec((BM, NUM_IN), lambda i: (i, 0)),
            pl.BlockSpec((BATCH, NUM_IN), lambda i: (0, 0)),
            pl.BlockSpec((BM, 1), lambda i: (i, 0)),
        ],
        out_specs=pl.BlockSpec((BM, BATCH), lambda i: (i, 0)),
        out_shape=jax.ShapeDtypeStruct((NUM_MID, BATCH), jnp.float32),
    )(W_in, x, b_in.reshape(NUM_MID, 1))


def _mm2_body(h_ref, w_ref, b_ref, o_ref):
    acc = lax.dot_general(h_ref[...], w_ref[...], (((0,), (1,)), ((), ())),
                          preferred_element_type=jnp.float32)
    o_ref[...] = acc + b_ref[...]


def _mm2(h2T, W_out, b_out):
    BO = 256
    return pl.pallas_call(
        _mm2_body,
        grid=(NUM_OUT // BO,),
        in_specs=[
            pl.BlockSpec((NUM_MID, BATCH), lambda i: (0, 0)),
            pl.BlockSpec((BO, NUM_MID), lambda i: (i, 0)),
            pl.BlockSpec((1, BO), lambda i: (0, i)),
        ],
        out_specs=pl.BlockSpec((BATCH, BO), lambda i: (0, i)),
        out_shape=jax.ShapeDtypeStruct((BATCH, NUM_OUT), jnp.float32),
    )(h2T, W_out, b_out.reshape(1, NUM_OUT))


def _cond_sc(hT, idx_f, wrep_f, brep_f, interpret=False):
    mesh = plsc.VectorSubcoreMesh(core_axis_name="c", subcore_axis_name="s",
                                  num_cores=NC, num_subcores=NS)

    @functools.partial(
        pl.kernel,
        out_type=jax.ShapeDtypeStruct((NUM_MID, BATCH), jnp.float32),
        mesh=mesh,
        interpret=interpret,
        scratch_types=[
            pltpu.VMEM((RPW * FAN_IN,), jnp.int32),
            pltpu.VMEM((RPW * FAN_IN * L,), jnp.float32),
            pltpu.VMEM((RPW * L,), jnp.float32),
            pltpu.VMEM((FAN_IN, BATCH), jnp.float32),
            pltpu.VMEM((BATCH,), jnp.float32),
            pltpu.SemaphoreType.DMA,
        ],
    )
    def k(hT_hbm, idx_hbm, wrep_hbm, brep_hbm, out_hbm,
          idx_v, wrep_v, brep_v, rows_v, orow_v, sem):
        wid = lax.axis_index("s") * NC + lax.axis_index("c")
        base = wid * RPW
        pltpu.sync_copy(idx_hbm.at[pl.ds(base * FAN_IN, RPW * FAN_IN)], idx_v)
        pltpu.sync_copy(wrep_hbm.at[pl.ds(base * FAN_IN * L, RPW * FAN_IN * L)],
                        wrep_v)
        pltpu.sync_copy(brep_hbm.at[pl.ds(base * L, RPW * L)], brep_v)

        def row_body(j, carry):
            idxrow = idx_v[pl.ds(j * FAN_IN, FAN_IN)]
            pltpu.async_copy(hT_hbm.at[idxrow], rows_v, sem).wait()
            bvec = brep_v[pl.ds(j * L, L)]
            wks = [wrep_v[pl.ds((j * FAN_IN + k) * L, L)]
                   for k in range(FAN_IN)]

            def chunk_body(c, carry2):
                acc = bvec
                for k in range(FAN_IN):
                    acc = acc + wks[k] * rows_v[k, pl.ds(c * L, L)]
                orow_v[pl.ds(c * L, L)] = jnp.maximum(acc, 0.0)
                return carry2

            lax.fori_loop(0, NCHUNK, chunk_body, 0)
            pltpu.sync_copy(orow_v, out_hbm.at[base + j])
            return carry

        lax.fori_loop(0, RPW, row_body, 0)

    return k(hT, idx_f, wrep_f, brep_f)


def kernel(x, W_in, b_in, W_mid0, b_mid0, W_mid1, b_mid1, W_out, b_out,
           indx_seqs):
    wrep0 = jnp.broadcast_to(W_mid0[:, :, None],
                             (NUM_MID, FAN_IN, L)).reshape(-1)
    brep0 = jnp.broadcast_to(b_mid0[:, None], (NUM_MID, L)).reshape(-1)
    wrep1 = jnp.broadcast_to(W_mid1[:, :, None],
                             (NUM_MID, FAN_IN, L)).reshape(-1)
    brep1 = jnp.broadcast_to(b_mid1[:, None], (NUM_MID, L)).reshape(-1)
    idx_f = indx_seqs.reshape(-1)

    h0T = _mm1(W_in, x, b_in)
    h1T = _cond_sc(h0T, idx_f, wrep0, brep0)
    h2T = _cond_sc(h1T, idx_f, wrep1, brep1)
    return _mm2(h2T, W_out, b_out)


# speedup vs baseline: 1.9610x; 1.6476x over previous
import functools

import jax
import jax.numpy as jnp
from jax import lax
from jax.experimental import pallas as pl
from jax.experimental.pallas import tpu as pltpu
from jax.experimental.pallas import tpu_sc as plsc

NUM_IN = 1024
NUM_OUT = 1024
NUM_MID = 4096
FAN_IN = 16
BATCH = 1024

NC = 2
NS = 16
NW = NC * NS
RPW = NUM_MID // NW
L = 16
NCHUNK = BATCH // L


def _mm1_body(w_ref, x_ref, b_ref, o_ref):
    acc = lax.dot_general(w_ref[...], x_ref[...], (((1,), (1,)), ((), ())),
                          preferred_element_type=jnp.float32)
    o_ref[...] = jnp.maximum(acc + b_ref[...], 0.0)


def _mm1(W_in, x, b_in):
    BM = 1024
    return pl.pallas_call(
        _mm1_body,
        grid=(NUM_MID // BM,),
        in_specs=[
            pl.BlockSpec((BM, NUM_IN), lambda i: (i, 0)),
            pl.BlockSpec((BATCH, NUM_IN), lambda i: (0, 0)),
            pl.BlockSpec((BM, 1), lambda i: (i, 0)),
        ],
        out_specs=pl.BlockSpec((BM, BATCH), lambda i: (i, 0)),
        out_shape=jax.ShapeDtypeStruct((NUM_MID, BATCH), jnp.float32),
    )(W_in, x, b_in.reshape(NUM_MID, 1))


def _mm2_body(h_ref, w_ref, b_ref, o_ref):
    acc = lax.dot_general(h_ref[...], w_ref[...], (((0,), (1,)), ((), ())),
                          preferred_element_type=jnp.float32)
    o_ref[...] = acc + b_ref[...]


def _mm2(h2T, W_out, b_out):
    BO = 256
    return pl.pallas_call(
        _mm2_body,
        grid=(NUM_OUT // BO,),
        in_specs=[
            pl.BlockSpec((NUM_MID, BATCH), lambda i: (0, 0)),
            pl.BlockSpec((BO, NUM_MID), lambda i: (i, 0)),
            pl.BlockSpec((1, BO), lambda i: (0, i)),
        ],
        out_specs=pl.BlockSpec((BATCH, BO), lambda i: (0, i)),
        out_shape=jax.ShapeDtypeStruct((BATCH, NUM_OUT), jnp.float32),
    )(h2T, W_out, b_out.reshape(1, NUM_OUT))


def _cond_sc(hT, idx_f, wrep_f, brep_f, interpret=False):
    mesh = plsc.VectorSubcoreMesh(core_axis_name="c", subcore_axis_name="s",
                                  num_cores=NC, num_subcores=NS)

    NBUF = 4

    @functools.partial(
        pl.kernel,
        out_type=jax.ShapeDtypeStruct((NUM_MID, BATCH), jnp.float32),
        mesh=mesh,
        interpret=interpret,
        scratch_types=[
            pltpu.VMEM((RPW * FAN_IN,), jnp.int32),
            pltpu.VMEM((RPW * FAN_IN * L,), jnp.float32),
            pltpu.VMEM((RPW * L,), jnp.float32),
            pltpu.VMEM((NBUF, FAN_IN, BATCH), jnp.float32),
            pltpu.VMEM((NBUF, BATCH), jnp.float32),
            [pltpu.SemaphoreType.DMA] * NBUF,
            [pltpu.SemaphoreType.DMA] * NBUF,
        ],
    )
    def k(hT_hbm, idx_hbm, wrep_hbm, brep_hbm, out_hbm,
          idx_v, wrep_v, brep_v, rows_v, ostage_v, gsems, osems):
        wid = lax.axis_index("s") * NC + lax.axis_index("c")
        base = wid * RPW
        pltpu.sync_copy(idx_hbm.at[pl.ds(base * FAN_IN, RPW * FAN_IN)], idx_v)
        pltpu.sync_copy(wrep_hbm.at[pl.ds(base * FAN_IN * L, RPW * FAN_IN * L)],
                        wrep_v)
        pltpu.sync_copy(brep_hbm.at[pl.ds(base * L, RPW * L)], brep_v)

        def gather_idx(j):
            return idx_v[pl.ds(j * FAN_IN, FAN_IN)]

        for b in range(NBUF):
            pltpu.async_copy(hT_hbm.at[gather_idx(b)], rows_v.at[b], gsems[b])

        def grp_body(j0, carry):
            for b in range(NBUF):
                j = j0 + b
                pltpu.make_async_copy(hT_hbm.at[gather_idx(j)], rows_v.at[b],
                                      gsems[b]).wait()
                @pl.when(j0 >= NBUF)
                def _():
                    pltpu.make_async_copy(ostage_v.at[b], out_hbm.at[base],
                                          osems[b]).wait()

                bvec = brep_v[pl.ds(j * L, L)]
                wks = [wrep_v[pl.ds((j * FAN_IN + k) * L, L)]
                       for k in range(FAN_IN)]

                def chunk_body(c, carry2):
                    acc = bvec
                    for k in range(FAN_IN):
                        acc = acc + wks[k] * rows_v[b, k, pl.ds(c * L, L)]
                    ostage_v[b, pl.ds(c * L, L)] = jnp.maximum(acc, 0.0)
                    return carry2

                lax.fori_loop(0, NCHUNK, chunk_body, 0)

                @pl.when(j0 < RPW - NBUF)
                def _():
                    pltpu.async_copy(hT_hbm.at[gather_idx(j + NBUF)],
                                     rows_v.at[b], gsems[b])

                pltpu.async_copy(ostage_v.at[b], out_hbm.at[base + j],
                                 osems[b])
            return carry

        lax.fori_loop(0, RPW // NBUF, lambda i, c: grp_body(i * NBUF, c), 0)
        for b in range(NBUF):
            pltpu.make_async_copy(ostage_v.at[b], out_hbm.at[base],
                                  osems[b]).wait()

    return k(hT, idx_f, wrep_f, brep_f)


def kernel(x, W_in, b_in, W_mid0, b_mid0, W_mid1, b_mid1, W_out, b_out,
           indx_seqs):
    wrep0 = jnp.broadcast_to(W_mid0[:, :, None],
                             (NUM_MID, FAN_IN, L)).reshape(-1)
    brep0 = jnp.broadcast_to(b_mid0[:, None], (NUM_MID, L)).reshape(-1)
    wrep1 = jnp.broadcast_to(W_mid1[:, :, None],
                             (NUM_MID, FAN_IN, L)).reshape(-1)
    brep1 = jnp.broadcast_to(b_mid1[:, None], (NUM_MID, L)).reshape(-1)
    idx_f = indx_seqs.reshape(-1)

    h0T = _mm1(W_in, x, b_in)
    h1T = _cond_sc(h0T, idx_f, wrep0, brep0)
    h2T = _cond_sc(h1T, idx_f, wrep1, brep1)
    return _mm2(h2T, W_out, b_out)


# speedup vs baseline: 3.0892x; 1.5753x over previous
import functools

import jax
import jax.numpy as jnp
from jax import lax
from jax.experimental import pallas as pl
from jax.experimental.pallas import tpu as pltpu
from jax.experimental.pallas import tpu_sc as plsc

NUM_IN = 1024
NUM_OUT = 1024
NUM_MID = 4096
FAN_IN = 16
BATCH = 1024

NC = 2
NS = 16
NW = NC * NS
RPW = NUM_MID // NW
L = 16
NCHUNK = BATCH // L


def _mm1_body(w_ref, x_ref, b_ref, o_ref):
    acc = lax.dot_general(w_ref[...], x_ref[...], (((1,), (1,)), ((), ())),
                          preferred_element_type=jnp.float32)
    o_ref[...] = jnp.maximum(acc + b_ref[...], 0.0)


def _mm1(W_in, x, b_in):
    BM = 1024
    return pl.pallas_call(
        _mm1_body,
        grid=(NUM_MID // BM,),
        in_specs=[
            pl.BlockSpec((BM, NUM_IN), lambda i: (i, 0)),
            pl.BlockSpec((BATCH, NUM_IN), lambda i: (0, 0)),
            pl.BlockSpec((BM, 1), lambda i: (i, 0)),
        ],
        out_specs=pl.BlockSpec((BM, BATCH), lambda i: (i, 0)),
        out_shape=jax.ShapeDtypeStruct((NUM_MID, BATCH), jnp.float32),
    )(W_in, x, b_in.reshape(NUM_MID, 1))


def _mm2_body(h_ref, w_ref, b_ref, o_ref):
    acc = lax.dot_general(h_ref[...], w_ref[...], (((0,), (1,)), ((), ())),
                          preferred_element_type=jnp.float32)
    o_ref[...] = acc + b_ref[...]


def _mm2(h2T, W_out, b_out):
    BO = 256
    return pl.pallas_call(
        _mm2_body,
        grid=(NUM_OUT // BO,),
        in_specs=[
            pl.BlockSpec((NUM_MID, BATCH), lambda i: (0, 0)),
            pl.BlockSpec((BO, NUM_MID), lambda i: (i, 0)),
            pl.BlockSpec((1, BO), lambda i: (0, i)),
        ],
        out_specs=pl.BlockSpec((BATCH, BO), lambda i: (0, i)),
        out_shape=jax.ShapeDtypeStruct((BATCH, NUM_OUT), jnp.float32),
    )(h2T, W_out, b_out.reshape(1, NUM_OUT))


def _cond_sc(hT, idx_f, wrep_f, brep_f, interpret=False):
    mesh = plsc.VectorSubcoreMesh(core_axis_name="c", subcore_axis_name="s",
                                  num_cores=NC, num_subcores=NS)

    NBUF = 4

    @functools.partial(
        pl.kernel,
        out_type=jax.ShapeDtypeStruct((NUM_MID, BATCH), jnp.float32),
        mesh=mesh,
        interpret=interpret,
        scratch_types=[
            pltpu.VMEM((RPW * FAN_IN,), jnp.int32),
            pltpu.VMEM((RPW * FAN_IN * L,), jnp.float32),
            pltpu.VMEM((RPW * L,), jnp.float32),
            pltpu.VMEM((NBUF, FAN_IN, BATCH), jnp.float32),
            pltpu.VMEM((NBUF, BATCH), jnp.float32),
            [pltpu.SemaphoreType.DMA] * NBUF,
            [pltpu.SemaphoreType.DMA] * NBUF,
        ],
    )
    def k(hT_hbm, idx_hbm, wrep_hbm, brep_hbm, out_hbm,
          idx_v, wrep_v, brep_v, rows_v, ostage_v, gsems, osems):
        wid = lax.axis_index("s") * NC + lax.axis_index("c")
        base = wid * RPW
        pltpu.sync_copy(idx_hbm.at[pl.ds(base * FAN_IN, RPW * FAN_IN)], idx_v)
        pltpu.sync_copy(wrep_hbm.at[pl.ds(base * FAN_IN * L, RPW * FAN_IN * L)],
                        wrep_v)
        pltpu.sync_copy(brep_hbm.at[pl.ds(base * L, RPW * L)], brep_v)

        def gather_idx(j):
            return idx_v[pl.ds(j * FAN_IN, FAN_IN)]

        for b in range(NBUF):
            pltpu.async_copy(hT_hbm.at[gather_idx(b)], rows_v.at[b], gsems[b])

        def grp_body(j0, carry):
            for b in range(NBUF):
                j = j0 + b
                pltpu.make_async_copy(hT_hbm.at[gather_idx(j)], rows_v.at[b],
                                      gsems[b]).wait()
                @pl.when(j0 >= NBUF)
                def _():
                    pltpu.make_async_copy(ostage_v.at[b], out_hbm.at[base],
                                          osems[b]).wait()

                bvec = brep_v[pl.ds(j * L, L)]
                wks = [wrep_v[pl.ds((j * FAN_IN + k) * L, L)]
                       for k in range(FAN_IN)]

                @plsc.parallel_loop(0, NCHUNK, unroll=4)
                def _(c):
                    acc = bvec
                    for k in range(FAN_IN):
                        acc = acc + wks[k] * rows_v[b, k, pl.ds(c * L, L)]
                    ostage_v[b, pl.ds(c * L, L)] = jnp.maximum(acc, 0.0)

                @pl.when(j0 < RPW - NBUF)
                def _():
                    pltpu.async_copy(hT_hbm.at[gather_idx(j + NBUF)],
                                     rows_v.at[b], gsems[b])

                pltpu.async_copy(ostage_v.at[b], out_hbm.at[base + j],
                                 osems[b])
            return carry

        lax.fori_loop(0, RPW // NBUF, lambda i, c: grp_body(i * NBUF, c), 0)
        for b in range(NBUF):
            pltpu.make_async_copy(ostage_v.at[b], out_hbm.at[base],
                                  osems[b]).wait()

    return k(hT, idx_f, wrep_f, brep_f)


def kernel(x, W_in, b_in, W_mid0, b_mid0, W_mid1, b_mid1, W_out, b_out,
           indx_seqs):
    wrep0 = jnp.broadcast_to(W_mid0[:, :, None],
                             (NUM_MID, FAN_IN, L)).reshape(-1)
    brep0 = jnp.broadcast_to(b_mid0[:, None], (NUM_MID, L)).reshape(-1)
    wrep1 = jnp.broadcast_to(W_mid1[:, :, None],
                             (NUM_MID, FAN_IN, L)).reshape(-1)
    brep1 = jnp.broadcast_to(b_mid1[:, None], (NUM_MID, L)).reshape(-1)
    idx_f = indx_seqs.reshape(-1)

    h0T = _mm1(W_in, x, b_in)
    h1T = _cond_sc(h0T, idx_f, wrep0, brep0)
    h2T = _cond_sc(h1T, idx_f, wrep1, brep1)
    return _mm2(h2T, W_out, b_out)
